# Initial kernel scaffold; baseline (speedup 1.0000x reference)
import jax
import jax.numpy as jnp
from jax.experimental import pallas as pl


def kernel(x, edge_index, W1, att_src1, att_dst1, bias1, W2, att_src2, att_dst2, bias2):
    raise NotImplementedError("write your pallas kernel here")


# speedup vs baseline: 45.5026x; 45.5026x over previous
import functools

import jax
import jax.numpy as jnp
from jax import lax
from jax.experimental import pallas as pl
from jax.experimental.pallas import tpu as pltpu
from jax.experimental.pallas import tpu_sc as plsc

N = 10000
IN = 128
HID = 16
HEADS = 8
OUT = 64
D1 = HEADS * HID

NC = 2
NS = 16
NW = NC * NS
L = 16

NP = 10112
RPT = NP // NS
B = 128
E_TOT = 320000 + N
CHUNK = NW * B
NBLK = -(-E_TOT // CHUNK)
EP = NBLK * CHUNK
BN = 1000


def _tc1_body(x_ref, w1_ref, as_ref, ad_ref, h_ref, a_s_ref, a_d_ref):
    h = jnp.dot(x_ref[...], w1_ref[...], preferred_element_type=jnp.float32)
    h_ref[...] = h
    a_s_ref[...] = jnp.dot(h, as_ref[...], preferred_element_type=jnp.float32)
    a_d_ref[...] = jnp.dot(h, ad_ref[...], preferred_element_type=jnp.float32)


def _tc1(x, W1, AS16, AD16):
    return pl.pallas_call(
        _tc1_body,
        grid=(N // BN,),
        in_specs=[
            pl.BlockSpec((BN, IN), lambda i: (i, 0)),
            pl.BlockSpec((IN, D1), lambda i: (0, 0)),
            pl.BlockSpec((D1, 16), lambda i: (0, 0)),
            pl.BlockSpec((D1, 16), lambda i: (0, 0)),
        ],
        out_specs=[
            pl.BlockSpec((BN, D1), lambda i: (i, 0)),
            pl.BlockSpec((BN, 16), lambda i: (i, 0)),
            pl.BlockSpec((BN, 16), lambda i: (i, 0)),
        ],
        out_shape=[
            jax.ShapeDtypeStruct((N, D1), jnp.float32),
            jax.ShapeDtypeStruct((N, 16), jnp.float32),
            jax.ShapeDtypeStruct((N, 16), jnp.float32),
        ],
    )(x, W1, AS16, AD16)


def _tc2_body(a0_ref, a1_ref, d0_ref, d1_ref, r_ref, b1_ref, w2_ref,
              ps_ref, pd_ref, h2_ref, a_s_ref, a_d_ref):
    den = d0_ref[...] + d1_ref[...]
    dfull = jnp.dot(den, r_ref[...], preferred_element_type=jnp.float32)
    g = (a0_ref[...] + a1_ref[...]) / (dfull + 1e-16) + b1_ref[...]
    hcur = jnp.where(g > 0.0, g, jnp.exp(g) - 1.0)
    h2 = jnp.dot(hcur, w2_ref[...], preferred_element_type=jnp.float32)
    h2_ref[...] = h2
    a_s_ref[...] = jnp.dot(h2, ps_ref[...], preferred_element_type=jnp.float32)
    a_d_ref[...] = jnp.dot(h2, pd_ref[...], preferred_element_type=jnp.float32)


def _tc2(a0, a1, d0, d1, R, b1, W2, PS, PD):
    return pl.pallas_call(
        _tc2_body,
        grid=(N // BN,),
        in_specs=[
            pl.BlockSpec((BN, D1), lambda i: (i, 0)),
            pl.BlockSpec((BN, D1), lambda i: (i, 0)),
            pl.BlockSpec((BN, 16), lambda i: (i, 0)),
            pl.BlockSpec((BN, 16), lambda i: (i, 0)),
            pl.BlockSpec((16, D1), lambda i: (0, 0)),
            pl.BlockSpec((1, D1), lambda i: (0, 0)),
            pl.BlockSpec((D1, OUT), lambda i: (0, 0)),
            pl.BlockSpec((OUT, 16), lambda i: (0, 0)),
            pl.BlockSpec((OUT, 16), lambda i: (0, 0)),
        ],
        out_specs=[
            pl.BlockSpec((BN, OUT), lambda i: (i, 0)),
            pl.BlockSpec((BN, 16), lambda i: (i, 0)),
            pl.BlockSpec((BN, 16), lambda i: (i, 0)),
        ],
        out_shape=[
            jax.ShapeDtypeStruct((N, OUT), jnp.float32),
            jax.ShapeDtypeStruct((N, 16), jnp.float32),
            jax.ShapeDtypeStruct((N, 16), jnp.float32),
        ],
    )(a0, a1, d0, d1, R, b1, W2, PS, PD)


def _tc3_body(a0_ref, a1_ref, d0_ref, d1_ref, q_ref, b2_ref, out_ref):
    den = jnp.dot(d0_ref[...] + d1_ref[...], q_ref[...],
                  preferred_element_type=jnp.float32)
    t = (a0_ref[...] + a1_ref[...]) / (den + 1e-16) + b2_ref[...]
    m = jnp.max(t, axis=1, keepdims=True)
    ex = jnp.exp(t - m)
    lse = jnp.log(jnp.sum(ex, axis=1, keepdims=True))
    out_ref[...] = t - m - lse


def _tc3(a0, a1, d0, d1, Q, b2):
    return pl.pallas_call(
        _tc3_body,
        grid=(N // BN,),
        in_specs=[
            pl.BlockSpec((BN, OUT), lambda i: (i, 0)),
            pl.BlockSpec((BN, OUT), lambda i: (i, 0)),
            pl.BlockSpec((BN, 16), lambda i: (i, 0)),
            pl.BlockSpec((BN, 16), lambda i: (i, 0)),
            pl.BlockSpec((16, OUT), lambda i: (0, 0)),
            pl.BlockSpec((1, OUT), lambda i: (0, 0)),
        ],
        out_specs=pl.BlockSpec((BN, OUT), lambda i: (i, 0)),
        out_shape=jax.ShapeDtypeStruct((N, OUT), jnp.float32),
    )(a0, a1, d0, d1, Q, b2)


def _make_sc_edge(D, H, name):
    CH = D // H
    mesh = plsc.VectorSubcoreMesh(
        core_axis_name="c", subcore_axis_name="s",
        num_cores=NC, num_subcores=NS)

    def body(h_hbm, as_hbm, ad_hbm, src_hbm, dst_hbm, zD_hbm, z16_hbm,
             acc_out, den_out,
             sidx, didx, gs, gd, hbuf, exbuf, acc_sh, den_sh, s1, s2, s3):
        c = lax.axis_index("c")
        s = lax.axis_index("s")
        r0 = s * RPT
        pltpu.sync_copy(zD_hbm.at[pl.ds(r0, RPT)], acc_sh.at[pl.ds(r0, RPT)])
        pltpu.sync_copy(z16_hbm.at[pl.ds(r0, RPT)], den_sh.at[pl.ds(r0, RPT)])
        plsc.subcore_barrier()

        wid = c * NS + s
        base0 = wid * (NBLK * B)
        lane = lax.broadcasted_iota(jnp.int32, (L,), 0)

        def blk(b, carry):
            base = base0 + b * B
            pltpu.sync_copy(src_hbm.at[pl.ds(base, B)], sidx)
            pltpu.sync_copy(dst_hbm.at[pl.ds(base, B)], didx)
            cp1 = pltpu.async_copy(as_hbm.at[sidx], gs, s1)
            cp2 = pltpu.async_copy(ad_hbm.at[didx], gd, s2)
            cp3 = pltpu.async_copy(h_hbm.at[sidx], hbuf, s3)
            cp1.wait()
            cp2.wait()
            cp3.wait()

            def edge(e, cy):
                u = gs[e, :] + gd[e, :]
                a = jnp.where(u >= 0.0, u, 0.2 * u)
                exm = jnp.where(lane < H, jnp.exp(a), 0.0)
                exbuf[e, :] = exm
                for hd in range(H):
                    scv = jnp.full((L,), exm[hd], dtype=jnp.float32)
                    for v in range(CH // L):
                        col = hd * CH + v * L
                        hbuf[e, pl.ds(col, L)] = hbuf[e, pl.ds(col, L)] * scv
                return cy

            lax.fori_loop(0, B, edge, 0)
            pltpu.sync_copy(exbuf, den_sh.at[didx], add=True)
            pltpu.sync_copy(hbuf, acc_sh.at[didx], add=True)
            return carry

        lax.fori_loop(0, NBLK, blk, 0)
        plsc.subcore_barrier()
        pltpu.sync_copy(acc_sh.at[pl.ds(r0, RPT)], acc_out.at[c, pl.ds(r0, RPT)])
        pltpu.sync_copy(den_sh.at[pl.ds(r0, RPT)], den_out.at[c, pl.ds(r0, RPT)])

    return pl.kernel(
        body,
        out_type=(jax.ShapeDtypeStruct((NC, NP, D), jnp.float32),
                  jax.ShapeDtypeStruct((NC, NP, 16), jnp.float32)),
        mesh=mesh,
        scratch_types=[
            pltpu.VMEM((B,), jnp.int32),
            pltpu.VMEM((B,), jnp.int32),
            pltpu.VMEM((B, 16), jnp.float32),
            pltpu.VMEM((B, 16), jnp.float32),
            pltpu.VMEM((B, D), jnp.float32),
            pltpu.VMEM((B, 16), jnp.float32),
            pltpu.VMEM_SHARED((NP, D), jnp.float32),
            pltpu.VMEM_SHARED((NP, 16), jnp.float32),
            pltpu.SemaphoreType.DMA,
            pltpu.SemaphoreType.DMA,
            pltpu.SemaphoreType.DMA,
        ],
        compiler_params=pltpu.CompilerParams(use_tc_tiling_on_sc=False),
        name=name,
    )


_sc_edge1 = _make_sc_edge(D1, HEADS, "gat_edge_l1")
_sc_edge2 = _make_sc_edge(OUT, 1, "gat_edge_l2")


def kernel(x, edge_index, W1, att_src1, att_dst1, bias1,
           W2, att_src2, att_dst2, bias2):
    f32 = jnp.float32
    loop = jnp.arange(N, dtype=jnp.int32)
    padv = jnp.full((EP - E_TOT,), N, dtype=jnp.int32)
    src = jnp.concatenate([edge_index[0], loop, padv])
    dst = jnp.concatenate([edge_index[1], loop, padv])

    af_s = att_src1.reshape(-1)
    af_d = att_dst1.reshape(-1)
    colh = jnp.arange(16)[None, :]
    rowh = (jnp.arange(D1) // HID)[:, None]
    AS16 = jnp.where(colh == rowh, af_s[:, None], 0.0).astype(f32)
    AD16 = jnp.where(colh == rowh, af_d[:, None], 0.0).astype(f32)
    R = jnp.where((jnp.arange(D1)[None, :] // HID) == jnp.arange(16)[:, None],
                  1.0, 0.0).astype(f32)
    PS = jnp.where(colh[:, :16] == 0, att_src2.reshape(-1)[:, None], 0.0).astype(f32)
    PD = jnp.where(colh[:, :16] == 0, att_dst2.reshape(-1)[:, None], 0.0).astype(f32)
    Q = jnp.where(jnp.arange(16)[:, None] == 0, jnp.ones((16, OUT), f32), 0.0)

    zD1 = jnp.zeros((NP, D1), f32)
    zD2 = jnp.zeros((NP, OUT), f32)
    z16 = jnp.zeros((NP, 16), f32)

    h1, a_s1, a_d1 = _tc1(x, W1, AS16, AD16)
    h1p = jnp.pad(h1, ((0, NP - N), (0, 0)))
    a_s1p = jnp.pad(a_s1, ((0, NP - N), (0, 0)))
    a_d1p = jnp.pad(a_d1, ((0, NP - N), (0, 0)))
    acc1, den1 = _sc_edge1(h1p, a_s1p, a_d1p, src, dst, zD1, z16)

    h2, a_s2, a_d2 = _tc2(acc1[0, :N], acc1[1, :N], den1[0, :N], den1[1, :N],
                          R, bias1.reshape(1, D1), W2, PS, PD)
    h2p = jnp.pad(h2, ((0, NP - N), (0, 0)))
    a_s2p = jnp.pad(a_s2, ((0, NP - N), (0, 0)))
    a_d2p = jnp.pad(a_d2, ((0, NP - N), (0, 0)))
    acc2, den2 = _sc_edge2(h2p, a_s2p, a_d2p, src, dst, zD2, z16)

    return _tc3(acc2[0, :N], acc2[1, :N], den2[0, :N], den2[1, :N],
                Q, bias2.reshape(1, OUT))


# speedup vs baseline: 65.9676x; 1.4498x over previous
import functools

import jax
import jax.numpy as jnp
from jax import lax
from jax.experimental import pallas as pl
from jax.experimental.pallas import tpu as pltpu
from jax.experimental.pallas import tpu_sc as plsc

N = 10000
IN = 128
HID = 16
HEADS = 8
OUT = 64
D1 = HEADS * HID

NC = 2
NS = 16
NW = NC * NS
L = 16

NP = 10112
RPT = NP // NS
B = 96
E_TOT = 320000 + N
CHUNK = NW * B
NBLK = 4 * (-(-E_TOT // (4 * CHUNK)))
EP = NBLK * CHUNK
EP_ARR = EP + 2 * B
BN = 1000


def _tc1_body(x_ref, w1_ref, as_ref, ad_ref, h_ref, a_s_ref, a_d_ref):
    h = jnp.dot(x_ref[...], w1_ref[...], preferred_element_type=jnp.float32)
    h_ref[...] = h
    a_s_ref[...] = jnp.dot(h, as_ref[...], preferred_element_type=jnp.float32)
    a_d_ref[...] = jnp.dot(h, ad_ref[...], preferred_element_type=jnp.float32)


def _tc1(x, W1, AS16, AD16):
    return pl.pallas_call(
        _tc1_body,
        grid=(N // BN,),
        in_specs=[
            pl.BlockSpec((BN, IN), lambda i: (i, 0)),
            pl.BlockSpec((IN, D1), lambda i: (0, 0)),
            pl.BlockSpec((D1, 16), lambda i: (0, 0)),
            pl.BlockSpec((D1, 16), lambda i: (0, 0)),
        ],
        out_specs=[
            pl.BlockSpec((BN, D1), lambda i: (i, 0)),
            pl.BlockSpec((BN, 16), lambda i: (i, 0)),
            pl.BlockSpec((BN, 16), lambda i: (i, 0)),
        ],
        out_shape=[
            jax.ShapeDtypeStruct((N, D1), jnp.float32),
            jax.ShapeDtypeStruct((N, 16), jnp.float32),
            jax.ShapeDtypeStruct((N, 16), jnp.float32),
        ],
    )(x, W1, AS16, AD16)


def _tc2_body(a0_ref, a1_ref, d0_ref, d1_ref, r_ref, b1_ref, w2_ref,
              ps_ref, pd_ref, h2_ref, a_s_ref, a_d_ref):
    den = d0_ref[...] + d1_ref[...]
    dfull = jnp.dot(den, r_ref[...], preferred_element_type=jnp.float32)
    g = (a0_ref[...] + a1_ref[...]) / (dfull + 1e-16) + b1_ref[...]
    hcur = jnp.where(g > 0.0, g, jnp.exp(g) - 1.0)
    h2 = jnp.dot(hcur, w2_ref[...], preferred_element_type=jnp.float32)
    h2_ref[...] = h2
    a_s_ref[...] = jnp.dot(h2, ps_ref[...], preferred_element_type=jnp.float32)
    a_d_ref[...] = jnp.dot(h2, pd_ref[...], preferred_element_type=jnp.float32)


def _tc2(a0, a1, d0, d1, R, b1, W2, PS, PD):
    return pl.pallas_call(
        _tc2_body,
        grid=(N // BN,),
        in_specs=[
            pl.BlockSpec((BN, D1), lambda i: (i, 0)),
            pl.BlockSpec((BN, D1), lambda i: (i, 0)),
            pl.BlockSpec((BN, 16), lambda i: (i, 0)),
            pl.BlockSpec((BN, 16), lambda i: (i, 0)),
            pl.BlockSpec((16, D1), lambda i: (0, 0)),
            pl.BlockSpec((1, D1), lambda i: (0, 0)),
            pl.BlockSpec((D1, OUT), lambda i: (0, 0)),
            pl.BlockSpec((OUT, 16), lambda i: (0, 0)),
            pl.BlockSpec((OUT, 16), lambda i: (0, 0)),
        ],
        out_specs=[
            pl.BlockSpec((BN, OUT), lambda i: (i, 0)),
            pl.BlockSpec((BN, 16), lambda i: (i, 0)),
            pl.BlockSpec((BN, 16), lambda i: (i, 0)),
        ],
        out_shape=[
            jax.ShapeDtypeStruct((N, OUT), jnp.float32),
            jax.ShapeDtypeStruct((N, 16), jnp.float32),
            jax.ShapeDtypeStruct((N, 16), jnp.float32),
        ],
    )(a0, a1, d0, d1, R, b1, W2, PS, PD)


def _tc3_body(a0_ref, a1_ref, d0_ref, d1_ref, q_ref, b2_ref, out_ref):
    den = jnp.dot(d0_ref[...] + d1_ref[...], q_ref[...],
                  preferred_element_type=jnp.float32)
    t = (a0_ref[...] + a1_ref[...]) / (den + 1e-16) + b2_ref[...]
    m = jnp.max(t, axis=1, keepdims=True)
    ex = jnp.exp(t - m)
    lse = jnp.log(jnp.sum(ex, axis=1, keepdims=True))
    out_ref[...] = t - m - lse


def _tc3(a0, a1, d0, d1, Q, b2):
    return pl.pallas_call(
        _tc3_body,
        grid=(N // BN,),
        in_specs=[
            pl.BlockSpec((BN, OUT), lambda i: (i, 0)),
            pl.BlockSpec((BN, OUT), lambda i: (i, 0)),
            pl.BlockSpec((BN, 16), lambda i: (i, 0)),
            pl.BlockSpec((BN, 16), lambda i: (i, 0)),
            pl.BlockSpec((16, OUT), lambda i: (0, 0)),
            pl.BlockSpec((1, OUT), lambda i: (0, 0)),
        ],
        out_specs=pl.BlockSpec((BN, OUT), lambda i: (i, 0)),
        out_shape=jax.ShapeDtypeStruct((N, OUT), jnp.float32),
    )(a0, a1, d0, d1, Q, b2)


def _make_sc_edge(D, H, name):
    CH = D // H
    mesh = plsc.VectorSubcoreMesh(
        core_axis_name="c", subcore_axis_name="s",
        num_cores=NC, num_subcores=NS)

    def body(h_hbm, as_hbm, ad_hbm, src_hbm, dst_hbm, zD_hbm, z16_hbm,
             acc_out, den_out, *rest):
        sidx = rest[0:4]
        didx = rest[4:8]
        gs = rest[8:10]
        gd = rest[10:12]
        hb = rest[12:14]
        exb = rest[14:16]
        acc_sh, den_sh = rest[16:18]
        gsem = (rest[18:21], rest[21:24])
        isem = rest[24:28]

        c = lax.axis_index("c")
        s = lax.axis_index("s")
        r0 = s * RPT
        wid = c * NS + s
        base0 = wid * (NBLK * B)
        lane = lax.broadcasted_iota(jnp.int32, (L,), 0)

        def idx_issue(b, q):
            base = base0 + b * B
            pltpu.async_copy(src_hbm.at[pl.ds(base, B)], sidx[q], isem[q])
            pltpu.async_copy(dst_hbm.at[pl.ds(base, B)], didx[q], isem[q])

        def idx_wait(b, q):
            base = base0 + b * B
            pltpu.make_async_copy(src_hbm.at[pl.ds(base, B)], sidx[q], isem[q]).wait()
            pltpu.make_async_copy(dst_hbm.at[pl.ds(base, B)], didx[q], isem[q]).wait()

        def g_issue(d, q):
            pltpu.async_copy(as_hbm.at[sidx[q]], gs[d], gsem[d][0])
            pltpu.async_copy(ad_hbm.at[didx[q]], gd[d], gsem[d][1])
            pltpu.async_copy(h_hbm.at[sidx[q]], hb[d], gsem[d][2])

        def g_wait(d, q):
            pltpu.make_async_copy(as_hbm.at[sidx[q]], gs[d], gsem[d][0]).wait()
            pltpu.make_async_copy(ad_hbm.at[didx[q]], gd[d], gsem[d][1]).wait()
            pltpu.make_async_copy(h_hbm.at[sidx[q]], hb[d], gsem[d][2]).wait()

        idx_issue(0, 0)
        idx_issue(1, 1)
        idx_wait(0, 0)
        g_issue(0, 0)

        pltpu.sync_copy(zD_hbm.at[pl.ds(r0, RPT)], acc_sh.at[pl.ds(r0, RPT)])
        pltpu.sync_copy(z16_hbm.at[pl.ds(r0, RPT)], den_sh.at[pl.ds(r0, RPT)])
        plsc.subcore_barrier()

        def quad(bb, carry):
            for p in range(4):
                b = bb * 4 + p
                d = p % 2
                dn = (p + 1) % 2
                qn = (p + 1) % 4
                qnn = (p + 2) % 4
                g_wait(d, p)
                idx_wait(b + 1, qn)
                g_issue(dn, qn)
                idx_issue(b + 2, qnn)
                gs_d, gd_d, hb_d, exb_d = gs[d], gd[d], hb[d], exb[d]

                def edge(e, cy):
                    u = gs_d[e, :] + gd_d[e, :]
                    a = jnp.where(u >= 0.0, u, 0.2 * u)
                    exm = jnp.where(lane < H, jnp.exp(a), 0.0)
                    exb_d[e, :] = exm
                    for hd in range(H):
                        scv = jnp.full((L,), exm[hd], dtype=jnp.float32)
                        for v in range(CH // L):
                            col = hd * CH + v * L
                            hb_d[e, pl.ds(col, L)] = hb_d[e, pl.ds(col, L)] * scv
                    return cy

                lax.fori_loop(0, B, edge, 0, unroll=4)
                pltpu.sync_copy(exb_d, den_sh.at[didx[p]], add=True)
                pltpu.sync_copy(hb_d, acc_sh.at[didx[p]], add=True)
            return carry

        lax.fori_loop(0, NBLK // 4, quad, 0)
        g_wait(0, 0)
        idx_wait(NBLK + 1, 1)
        plsc.subcore_barrier()
        pltpu.sync_copy(acc_sh.at[pl.ds(r0, RPT)], acc_out.at[c, pl.ds(r0, RPT)])
        pltpu.sync_copy(den_sh.at[pl.ds(r0, RPT)], den_out.at[c, pl.ds(r0, RPT)])

    return pl.kernel(
        body,
        out_type=(jax.ShapeDtypeStruct((NC, NP, D), jnp.float32),
                  jax.ShapeDtypeStruct((NC, NP, 16), jnp.float32)),
        mesh=mesh,
        scratch_types=(
            [pltpu.VMEM((B,), jnp.int32) for _ in range(8)]
            + [pltpu.VMEM((B, 16), jnp.float32) for _ in range(4)]
            + [pltpu.VMEM((B, D), jnp.float32) for _ in range(2)]
            + [pltpu.VMEM((B, 16), jnp.float32) for _ in range(2)]
            + [pltpu.VMEM_SHARED((NP, D), jnp.float32),
               pltpu.VMEM_SHARED((NP, 16), jnp.float32)]
            + [pltpu.SemaphoreType.DMA for _ in range(10)]
        ),
        compiler_params=pltpu.CompilerParams(use_tc_tiling_on_sc=False),
        name=name,
    )


_sc_edge1 = _make_sc_edge(D1, HEADS, "gat_edge_l1")
_sc_edge2 = _make_sc_edge(OUT, 1, "gat_edge_l2")


def kernel(x, edge_index, W1, att_src1, att_dst1, bias1,
           W2, att_src2, att_dst2, bias2):
    f32 = jnp.float32
    loop = jnp.arange(N, dtype=jnp.int32)
    padv = jnp.full((EP_ARR - E_TOT,), N, dtype=jnp.int32)
    src = jnp.concatenate([edge_index[0], loop, padv])
    dst = jnp.concatenate([edge_index[1], loop, padv])

    af_s = att_src1.reshape(-1)
    af_d = att_dst1.reshape(-1)
    colh = jnp.arange(16)[None, :]
    rowh = (jnp.arange(D1) // HID)[:, None]
    AS16 = jnp.where(colh == rowh, af_s[:, None], 0.0).astype(f32)
    AD16 = jnp.where(colh == rowh, af_d[:, None], 0.0).astype(f32)
    R = jnp.where((jnp.arange(D1)[None, :] // HID) == jnp.arange(16)[:, None],
                  1.0, 0.0).astype(f32)
    PS = jnp.where(colh[:, :16] == 0, att_src2.reshape(-1)[:, None], 0.0).astype(f32)
    PD = jnp.where(colh[:, :16] == 0, att_dst2.reshape(-1)[:, None], 0.0).astype(f32)
    Q = jnp.where(jnp.arange(16)[:, None] == 0, jnp.ones((16, OUT), f32), 0.0)

    zD1 = jnp.zeros((NP, D1), f32)
    zD2 = jnp.zeros((NP, OUT), f32)
    z16 = jnp.zeros((NP, 16), f32)

    h1, a_s1, a_d1 = _tc1(x, W1, AS16, AD16)
    h1p = jnp.pad(h1, ((0, NP - N), (0, 0)))
    a_s1p = jnp.pad(a_s1, ((0, NP - N), (0, 0)))
    a_d1p = jnp.pad(a_d1, ((0, NP - N), (0, 0)))
    acc1, den1 = _sc_edge1(h1p, a_s1p, a_d1p, src, dst, zD1, z16)

    h2, a_s2, a_d2 = _tc2(acc1[0, :N], acc1[1, :N], den1[0, :N], den1[1, :N],
                          R, bias1.reshape(1, D1), W2, PS, PD)
    h2p = jnp.pad(h2, ((0, NP - N), (0, 0)))
    a_s2p = jnp.pad(a_s2, ((0, NP - N), (0, 0)))
    a_d2p = jnp.pad(a_d2, ((0, NP - N), (0, 0)))
    acc2, den2 = _sc_edge2(h2p, a_s2p, a_d2p, src, dst, zD2, z16)

    return _tc3(acc2[0, :N], acc2[1, :N], den2[0, :N], den2[1, :N],
                Q, bias2.reshape(1, OUT))


# speedup vs baseline: 102.0240x; 1.5466x over previous
import functools

import jax
import jax.numpy as jnp
from jax import lax
from jax.experimental import pallas as pl
from jax.experimental.pallas import tpu as pltpu
from jax.experimental.pallas import tpu_sc as plsc

N = 10000
IN = 128
HID = 16
HEADS = 8
OUT = 64
D1 = HEADS * HID

NC = 2
NS = 16
NW = NC * NS
L = 16

NP = 10112
RPT = NP // NS
B = 96
E_TOT = 320000 + N
CHUNK = NW * B
NBLK = 4 * (-(-E_TOT // (4 * CHUNK)))
EP = NBLK * CHUNK
EP_ARR = EP + 2 * B
BN = 1000


def _tc1_body(x_ref, w1_ref, as_ref, ad_ref, h_ref, a_s_ref, a_d_ref):
    h = jnp.dot(x_ref[...], w1_ref[...], preferred_element_type=jnp.float32)
    h_ref[...] = h
    a_s_ref[...] = jnp.dot(h, as_ref[...], preferred_element_type=jnp.float32)
    a_d_ref[...] = jnp.dot(h, ad_ref[...], preferred_element_type=jnp.float32)


def _tc1(x, W1, AS16, AD16):
    return pl.pallas_call(
        _tc1_body,
        grid=(N // BN,),
        in_specs=[
            pl.BlockSpec((BN, IN), lambda i: (i, 0)),
            pl.BlockSpec((IN, D1), lambda i: (0, 0)),
            pl.BlockSpec((D1, 16), lambda i: (0, 0)),
            pl.BlockSpec((D1, 16), lambda i: (0, 0)),
        ],
        out_specs=[
            pl.BlockSpec((BN, D1), lambda i: (i, 0)),
            pl.BlockSpec((BN, 16), lambda i: (i, 0)),
            pl.BlockSpec((BN, 16), lambda i: (i, 0)),
        ],
        out_shape=[
            jax.ShapeDtypeStruct((NP, D1), jnp.float32),
            jax.ShapeDtypeStruct((NP, 16), jnp.float32),
            jax.ShapeDtypeStruct((NP, 16), jnp.float32),
        ],
    )(x, W1, AS16, AD16)


def _tc2_body(acc_ref, den_ref, r_ref, b1_ref, w2_ref,
              ps_ref, pd_ref, h2_ref, a_s_ref, a_d_ref):
    den = den_ref[0] + den_ref[1]
    dfull = jnp.dot(den, r_ref[...], preferred_element_type=jnp.float32)
    g = (acc_ref[0] + acc_ref[1]) / (dfull + 1e-16) + b1_ref[...]
    hcur = jnp.where(g > 0.0, g, jnp.exp(g) - 1.0)
    h2 = jnp.dot(hcur, w2_ref[...], preferred_element_type=jnp.float32)
    h2_ref[...] = h2
    a_s_ref[...] = jnp.dot(h2, ps_ref[...], preferred_element_type=jnp.float32)
    a_d_ref[...] = jnp.dot(h2, pd_ref[...], preferred_element_type=jnp.float32)


def _tc2(acc, den, R, b1, W2, PS, PD):
    return pl.pallas_call(
        _tc2_body,
        grid=(N // BN,),
        in_specs=[
            pl.BlockSpec((NC, BN, D1), lambda i: (0, i, 0)),
            pl.BlockSpec((NC, BN, 16), lambda i: (0, i, 0)),
            pl.BlockSpec((16, D1), lambda i: (0, 0)),
            pl.BlockSpec((1, D1), lambda i: (0, 0)),
            pl.BlockSpec((D1, OUT), lambda i: (0, 0)),
            pl.BlockSpec((OUT, 16), lambda i: (0, 0)),
            pl.BlockSpec((OUT, 16), lambda i: (0, 0)),
        ],
        out_specs=[
            pl.BlockSpec((BN, OUT), lambda i: (i, 0)),
            pl.BlockSpec((BN, 16), lambda i: (i, 0)),
            pl.BlockSpec((BN, 16), lambda i: (i, 0)),
        ],
        out_shape=[
            jax.ShapeDtypeStruct((NP, OUT), jnp.float32),
            jax.ShapeDtypeStruct((NP, 16), jnp.float32),
            jax.ShapeDtypeStruct((NP, 16), jnp.float32),
        ],
    )(acc, den, R, b1, W2, PS, PD)


def _tc3_body(acc_ref, den_ref, q_ref, b2_ref, out_ref):
    den = jnp.dot(den_ref[0] + den_ref[1], q_ref[...],
                  preferred_element_type=jnp.float32)
    t = (acc_ref[0] + acc_ref[1]) / (den + 1e-16) + b2_ref[...]
    m = jnp.max(t, axis=1, keepdims=True)
    ex = jnp.exp(t - m)
    lse = jnp.log(jnp.sum(ex, axis=1, keepdims=True))
    out_ref[...] = t - m - lse


def _tc3(acc, den, Q, b2):
    return pl.pallas_call(
        _tc3_body,
        grid=(N // BN,),
        in_specs=[
            pl.BlockSpec((NC, BN, OUT), lambda i: (0, i, 0)),
            pl.BlockSpec((NC, BN, 16), lambda i: (0, i, 0)),
            pl.BlockSpec((16, OUT), lambda i: (0, 0)),
            pl.BlockSpec((1, OUT), lambda i: (0, 0)),
        ],
        out_specs=pl.BlockSpec((BN, OUT), lambda i: (i, 0)),
        out_shape=jax.ShapeDtypeStruct((N, OUT), jnp.float32),
    )(acc, den, Q, b2)


def _make_sc_edge(D, H, name):
    CH = D // H
    mesh = plsc.VectorSubcoreMesh(
        core_axis_name="c", subcore_axis_name="s",
        num_cores=NC, num_subcores=NS)

    def body(h_hbm, as_hbm, ad_hbm, src_hbm, dst_hbm, zD_hbm, z16_hbm,
             acc_out, den_out, *rest):
        sidx = rest[0:4]
        didx = rest[4:8]
        gs = rest[8:10]
        gd = rest[10:12]
        hb = rest[12:14]
        exb = rest[14:16]
        acc_sh, den_sh = rest[16:18]
        gsem = (rest[18:21], rest[21:24])
        isem = rest[24:28]

        c = lax.axis_index("c")
        s = lax.axis_index("s")
        r0 = s * RPT
        wid = c * NS + s
        base0 = wid * (NBLK * B)
        lane = lax.broadcasted_iota(jnp.int32, (L,), 0)

        def idx_issue(b, q):
            base = base0 + b * B
            pltpu.async_copy(src_hbm.at[pl.ds(base, B)], sidx[q], isem[q])
            pltpu.async_copy(dst_hbm.at[pl.ds(base, B)], didx[q], isem[q])

        def idx_wait(b, q):
            base = base0 + b * B
            pltpu.make_async_copy(src_hbm.at[pl.ds(base, B)], sidx[q], isem[q]).wait()
            pltpu.make_async_copy(dst_hbm.at[pl.ds(base, B)], didx[q], isem[q]).wait()

        def g_issue(d, q):
            pltpu.async_copy(as_hbm.at[sidx[q]], gs[d], gsem[d][0])
            pltpu.async_copy(ad_hbm.at[didx[q]], gd[d], gsem[d][1])
            pltpu.async_copy(h_hbm.at[sidx[q]], hb[d], gsem[d][2])

        def g_wait(d, q):
            pltpu.make_async_copy(as_hbm.at[sidx[q]], gs[d], gsem[d][0]).wait()
            pltpu.make_async_copy(ad_hbm.at[didx[q]], gd[d], gsem[d][1]).wait()
            pltpu.make_async_copy(h_hbm.at[sidx[q]], hb[d], gsem[d][2]).wait()

        idx_issue(0, 0)
        idx_issue(1, 1)
        idx_wait(0, 0)
        g_issue(0, 0)

        pltpu.sync_copy(zD_hbm.at[pl.ds(r0, RPT)], acc_sh.at[pl.ds(r0, RPT)])
        pltpu.sync_copy(z16_hbm.at[pl.ds(r0, RPT)], den_sh.at[pl.ds(r0, RPT)])
        plsc.subcore_barrier()

        def quad(bb, carry):
            for p in range(4):
                b = bb * 4 + p
                d = p % 2
                dn = (p + 1) % 2
                qn = (p + 1) % 4
                qnn = (p + 2) % 4
                g_wait(d, p)
                idx_wait(b + 1, qn)
                g_issue(dn, qn)
                idx_issue(b + 2, qnn)
                gs_d, gd_d, hb_d, exb_d = gs[d], gd[d], hb[d], exb[d]

                @plsc.parallel_loop(0, B, unroll=4)
                def edge(e):
                    u = gs_d[e, :] + gd_d[e, :]
                    a = jnp.where(u >= 0.0, u, 0.2 * u)
                    exm = jnp.where(lane < H, jnp.exp(a), 0.0)
                    exb_d[e, :] = exm
                    for hd in range(H):
                        scv = jnp.full((L,), exm[hd], dtype=jnp.float32)
                        for v in range(CH // L):
                            col = hd * CH + v * L
                            hb_d[e, pl.ds(col, L)] = hb_d[e, pl.ds(col, L)] * scv
                pltpu.sync_copy(exb_d, den_sh.at[didx[p]], add=True)
                pltpu.sync_copy(hb_d, acc_sh.at[didx[p]], add=True)
            return carry

        lax.fori_loop(0, NBLK // 4, quad, 0)
        g_wait(0, 0)
        idx_wait(NBLK + 1, 1)
        plsc.subcore_barrier()
        pltpu.sync_copy(acc_sh.at[pl.ds(r0, RPT)], acc_out.at[c, pl.ds(r0, RPT)])
        pltpu.sync_copy(den_sh.at[pl.ds(r0, RPT)], den_out.at[c, pl.ds(r0, RPT)])

    return pl.kernel(
        body,
        out_type=(jax.ShapeDtypeStruct((NC, NP, D), jnp.float32),
                  jax.ShapeDtypeStruct((NC, NP, 16), jnp.float32)),
        mesh=mesh,
        scratch_types=(
            [pltpu.VMEM((B,), jnp.int32) for _ in range(8)]
            + [pltpu.VMEM((B, 16), jnp.float32) for _ in range(4)]
            + [pltpu.VMEM((B, D), jnp.float32) for _ in range(2)]
            + [pltpu.VMEM((B, 16), jnp.float32) for _ in range(2)]
            + [pltpu.VMEM_SHARED((NP, D), jnp.float32),
               pltpu.VMEM_SHARED((NP, 16), jnp.float32)]
            + [pltpu.SemaphoreType.DMA for _ in range(10)]
        ),
        compiler_params=pltpu.CompilerParams(use_tc_tiling_on_sc=False),
        name=name,
    )


_sc_edge1 = _make_sc_edge(D1, HEADS, "gat_edge_l1")
_sc_edge2 = _make_sc_edge(OUT, 1, "gat_edge_l2")


def kernel(x, edge_index, W1, att_src1, att_dst1, bias1,
           W2, att_src2, att_dst2, bias2):
    f32 = jnp.float32
    loop = jnp.arange(N, dtype=jnp.int32)
    padv = jnp.full((EP_ARR - E_TOT,), N, dtype=jnp.int32)
    src = jnp.concatenate([edge_index[0], loop, padv])
    dst = jnp.concatenate([edge_index[1], loop, padv])

    af_s = att_src1.reshape(-1)
    af_d = att_dst1.reshape(-1)
    colh = jnp.arange(16)[None, :]
    rowh = (jnp.arange(D1) // HID)[:, None]
    AS16 = jnp.where(colh == rowh, af_s[:, None], 0.0).astype(f32)
    AD16 = jnp.where(colh == rowh, af_d[:, None], 0.0).astype(f32)
    R = jnp.where((jnp.arange(D1)[None, :] // HID) == jnp.arange(16)[:, None],
                  1.0, 0.0).astype(f32)
    PS = jnp.where(colh[:, :16] == 0, att_src2.reshape(-1)[:, None], 0.0).astype(f32)
    PD = jnp.where(colh[:, :16] == 0, att_dst2.reshape(-1)[:, None], 0.0).astype(f32)
    Q = jnp.where(jnp.arange(16)[:, None] == 0, jnp.ones((16, OUT), f32), 0.0)

    zD1 = jnp.zeros((NP, D1), f32)
    zD2 = jnp.zeros((NP, OUT), f32)
    z16 = jnp.zeros((NP, 16), f32)

    h1, a_s1, a_d1 = _tc1(x, W1, AS16, AD16)
    acc1, den1 = _sc_edge1(h1, a_s1, a_d1, src, dst, zD1, z16)

    h2, a_s2, a_d2 = _tc2(acc1, den1, R, bias1.reshape(1, D1), W2, PS, PD)
    acc2, den2 = _sc_edge2(h2, a_s2, a_d2, src, dst, zD2, z16)

    return _tc3(acc2, den2, Q, bias2.reshape(1, OUT))
